# baseline (speedup 1.0000x reference)
import functools

import jax
import jax.numpy as jnp
from jax import lax
from jax.experimental import pallas as pl
from jax.experimental.pallas import tpu as pltpu
from jax.experimental.pallas import tpu_sc as plsc

_THETA = 40.0
_H = 128
_W = 128
_L = 16
_G = _W // _L
_B = 2
_C = 19


def _row(ref, h, g):
    return ref[h, pl.ds(g * _L, _L)]


def _acc_plane_init(x_ref, acc_ref):
    def body(h, _):
        for g in range(_G):
            acc_ref[h, pl.ds(g * _L, _L)] = -3.0 * _row(x_ref, h, g)
        return 0
    lax.fori_loop(0, _H, body, 0)


def _scan_h(x_ref, d_ref, acc_ref, forward):
    if forward:
        first, lo, hi = 0, 1, _H
    else:
        first, lo, hi = _H - 1, 1, _H

    f0 = tuple(_row(x_ref, first, g) for g in range(_G))
    for g in range(_G):
        plsc.addupdate(acc_ref.at[first, pl.ds(g * _L, _L)], f0[g])

    def body(i, carry):
        h = i if forward else _H - 1 - i
        dh = h if forward else h + 1
        out = []
        for g in range(_G):
            fv = _row(x_ref, h, g) + _row(d_ref, dh, g) * carry[g]
            plsc.addupdate(acc_ref.at[h, pl.ds(g * _L, _L)], fv)
            out.append(fv)
        return tuple(out)

    lax.fori_loop(lo, hi, body, f0)


def _scan_w(x_ref, d_ref, acc_ref, forward):
    iota = lax.iota(jnp.int32, _L)
    for hg in range(_G):
        h_idx = hg * _L + iota
        first = 0 if forward else _W - 1

        w0 = jnp.full((_L,), first, jnp.int32)
        f = plsc.load_gather(x_ref, [h_idx, w0])
        plsc.addupdate_scatter(acc_ref, [h_idx, w0], f)

        def body(i, carry, h_idx=h_idx):
            w = i if forward else _W - 1 - i
            dw = w if forward else w + 1
            w_idx = jnp.full((_L,), w, jnp.int32)
            dw_idx = jnp.full((_L,), dw, jnp.int32)
            xv = plsc.load_gather(x_ref, [h_idx, w_idx])
            dv = plsc.load_gather(d_ref, [h_idx, dw_idx])
            fv = xv + dv * carry
            plsc.addupdate_scatter(acc_ref, [h_idx, w_idx], fv)
            return fv

        lax.fori_loop(1, _W, body, f)


def _crisscross_into(x_ref, d_ref, acc_ref):
    _acc_plane_init(x_ref, acc_ref)
    _scan_h(x_ref, d_ref, acc_ref, True)
    _scan_h(x_ref, d_ref, acc_ref, False)
    _scan_w(x_ref, d_ref, acc_ref, True)
    _scan_w(x_ref, d_ref, acc_ref, False)


def _sc_body(x_hbm, e_hbm, it_hbm, out_hbm, xa, xb, d, rz, acc, it_vmem):
    c = lax.axis_index("c")
    s = lax.axis_index("s")

    pltpu.sync_copy(it_hbm, it_vmem)
    it = it_vmem[...][0]

    pltpu.sync_copy(e_hbm.at[c], d)

    def mk_d(h, _):
        for g in range(_G):
            v = _row(d, h, g)
            d[h, pl.ds(g * _L, _L)] = jnp.exp(-_THETA * jnp.maximum(v, 0.0))
        return 0
    lax.fori_loop(0, _H, body_fn := mk_d, 0)

    def mk_ones(h, _):
        for g in range(_G):
            rz[h, pl.ds(g * _L, _L)] = jnp.full((_L,), 1.0, jnp.float32)
        return 0
    lax.fori_loop(0, _H, mk_ones, 0)
    _crisscross_into(rz, d, acc)

    def mk_rz(h, _):
        for g in range(_G):
            rz[h, pl.ds(g * _L, _L)] = 1.0 / _row(acc, h, g)
        return 0
    lax.fori_loop(0, _H, mk_rz, 0)

    ch_a = c * _C + s
    pltpu.sync_copy(x_hbm.at[ch_a], xa)
    second = s < (_C - _L)
    ch_b = c * _C + _L + s

    @pl.when(second)
    def _():
        pltpu.sync_copy(x_hbm.at[ch_b], xb)

    def one_round(_, carry):
        _crisscross_into(xa, d, acc)

        def upd_a(h, _):
            for g in range(_G):
                xa[h, pl.ds(g * _L, _L)] = _row(acc, h, g) * _row(rz, h, g)
            return 0
        lax.fori_loop(0, _H, upd_a, 0)

        @pl.when(second)
        def _():
            _crisscross_into(xb, d, acc)

            def upd_b(h, _):
                for g in range(_G):
                    xb[h, pl.ds(g * _L, _L)] = _row(acc, h, g) * _row(rz, h, g)
                return 0
            lax.fori_loop(0, _H, upd_b, 0)

        return carry

    lax.fori_loop(0, it, one_round, 0)

    pltpu.sync_copy(xa, out_hbm.at[ch_a])

    @pl.when(second)
    def _():
        pltpu.sync_copy(xb, out_hbm.at[ch_b])


def kernel(mask, edge, iter):
    B, C, H, W = mask.shape
    x = mask.reshape(B * C, H, W)
    e = edge.reshape(B, H, W)
    it = jnp.broadcast_to(jnp.asarray(iter, jnp.int32), (16,))

    mesh = plsc.VectorSubcoreMesh(
        core_axis_name="c", subcore_axis_name="s",
        num_cores=2, num_subcores=16)
    run = pl.kernel(
        _sc_body,
        out_type=jax.ShapeDtypeStruct((B * C, H, W), jnp.float32),
        mesh=mesh,
        scratch_types=[
            pltpu.VMEM((H, W), jnp.float32),
            pltpu.VMEM((H, W), jnp.float32),
            pltpu.VMEM((H, W), jnp.float32),
            pltpu.VMEM((H, W), jnp.float32),
            pltpu.VMEM((H, W), jnp.float32),
            pltpu.VMEM((16,), jnp.int32),
        ],
        compiler_params=pltpu.CompilerParams(needs_layout_passes=False),
    )
    out = run(x, e, it)
    return out.reshape(B, C, H, W)


# speedup vs baseline: 13.1399x; 13.1399x over previous
import jax
import jax.numpy as jnp
from jax.experimental import pallas as pl
from jax.experimental.pallas import tpu as pltpu

_THETA = 40.0
_KS = (1, 2, 4, 8, 16, 32, 64)


def _shift_down(a, k, axis):
    n = a.shape[axis]
    zeros = jnp.zeros_like(jax.lax.slice_in_dim(a, 0, k, axis=axis))
    return jnp.concatenate(
        [zeros, jax.lax.slice_in_dim(a, 0, n - k, axis=axis)], axis=axis)


def _shift_up(a, k, axis):
    n = a.shape[axis]
    zeros = jnp.zeros_like(jax.lax.slice_in_dim(a, 0, k, axis=axis))
    return jnp.concatenate(
        [jax.lax.slice_in_dim(a, k, n, axis=axis), zeros], axis=axis)


def _build_ladder(d0, shift, axis):
    ladder = []
    dcur = d0
    for k in _KS:
        ladder.append(dcur)
        if k != _KS[-1]:
            dcur = dcur * shift(dcur, k, axis)
    return ladder


def _scan(x, ladder, shift, axis):
    f = x
    for dcur, k in zip(ladder, _KS):
        f = f + dcur * shift(f, k, axis)
    return f


def _ccedge_body(it_ref, mask_ref, edge_ref, out_ref):
    x0 = mask_ref[...]
    e = jnp.maximum(edge_ref[...], 0.0)
    d = jnp.exp(-_THETA * e)

    iota_h = jax.lax.broadcasted_iota(jnp.int32, d.shape, 2)
    iota_w = jax.lax.broadcasted_iota(jnp.int32, d.shape, 3)
    d0_fh = jnp.where(iota_h == 0, 0.0, d)
    d0_fw = jnp.where(iota_w == 0, 0.0, d)
    d0_bh = _shift_up(d, 1, 2)
    d0_bw = _shift_up(d, 1, 3)

    lad_fh = _build_ladder(d0_fh, _shift_down, 2)
    lad_bh = _build_ladder(d0_bh, _shift_up, 2)
    lad_fw = _build_ladder(d0_fw, _shift_down, 3)
    lad_bw = _build_ladder(d0_bw, _shift_up, 3)

    def crisscross(x):
        fh = _scan(x, lad_fh, _shift_down, 2)
        bh = _scan(x, lad_bh, _shift_up, 2)
        fw = _scan(x, lad_fw, _shift_down, 3)
        bw = _scan(x, lad_bw, _shift_up, 3)
        return fh + bh + fw + bw - 3.0 * x

    ones = jnp.ones_like(d)
    rz = 1.0 / crisscross(ones)

    def one_iter(_, x):
        return crisscross(x) * rz

    it = it_ref[0]
    x = x0
    for t in range(3):
        x = jnp.where(it > t, one_iter(t, x), x)
    out_ref[...] = jax.lax.fori_loop(3, jnp.maximum(it, 3), one_iter, x)


def kernel(mask, edge, iter):
    it = jnp.asarray(iter, jnp.int32).reshape(1)
    return pl.pallas_call(
        _ccedge_body,
        out_shape=jax.ShapeDtypeStruct(mask.shape, mask.dtype),
        in_specs=[
            pl.BlockSpec(memory_space=pltpu.SMEM),
            pl.BlockSpec(memory_space=pltpu.VMEM),
            pl.BlockSpec(memory_space=pltpu.VMEM),
        ],
        out_specs=pl.BlockSpec(memory_space=pltpu.VMEM),
    )(it, mask, edge)
